# baseline (speedup 1.0000x reference)
import functools

import jax
import jax.numpy as jnp
from jax import lax
from jax.experimental import pallas as pl
from jax.experimental.pallas import tpu as pltpu
from jax.experimental.pallas import tpu_sc as plsc

_N = 10000
_F = 128
_FX = 144
_NC = 2
_NS = 16
_NW = _NC * _NS
_C = 128
_B = 8
_NPT = _N // _NS

_DNUMS = lax.GatherDimensionNumbers(
    offset_dims=(), collapsed_slice_dims=(0,), start_index_map=(0,))


def _lane(vec, i):
    idx = jnp.full((16, 1), i, jnp.int32)
    return lax.gather(vec, idx, _DNUMS, (1,),
                      mode=lax.GatherScatterMode.PROMISE_IN_BOUNDS)


def _prep_body(x_ref, w_ref, asw_ref, anw_ref, featx_ref, asv_ref, anv_ref):
    feat = jnp.dot(x_ref[...], w_ref[...], preferred_element_type=jnp.float32)
    asv = jnp.dot(feat, asw_ref[...], preferred_element_type=jnp.float32)
    anv = jnp.dot(feat, anw_ref[...], preferred_element_type=jnp.float32)
    n = feat.shape[0]
    ext = jnp.concatenate(
        [feat, jnp.ones((n, 1), jnp.float32),
         jnp.zeros((n, _FX - _F - 1), jnp.float32)], axis=1)
    featx_ref[...] = ext
    asv_ref[...] = asv
    anv_ref[...] = anv


def _fin_body(p_ref, b_ref, o_ref):
    t = p_ref[0] + p_ref[1]
    numer = t[:, :_F]
    den = t[:, _F:_F + 1]
    o_ref[...] = jnp.maximum(numer / (den + 1e-9) + b_ref[...], 0.0)


def _make_agg(e_real, e_pad):
    ept = e_pad // _NW
    nchunk = ept // _C
    nbatch = nchunk // _B
    mesh = plsc.VectorSubcoreMesh(core_axis_name="c", subcore_axis_name="s",
                                  num_cores=_NC)

    @functools.partial(
        pl.kernel,
        out_type=[
            jax.ShapeDtypeStruct((_NC, _N, _FX), jnp.float32),
            jax.ShapeDtypeStruct((e_pad,), jnp.float32),
        ],
        mesh=mesh,
        compiler_params=pltpu.CompilerParams(use_tc_tiling_on_sc=False,
                                             needs_layout_passes=False),
        scratch_types=[
            pltpu.VMEM_SHARED((_N, _FX), jnp.float32),
            pltpu.SemaphoreType.DMA,
            pltpu.SemaphoreType.DMA,
            pltpu.SemaphoreType.DMA,
            pltpu.SemaphoreType.DMA,
        ])
    def agg(edge_hbm, asv_hbm, anv_hbm, featx_hbm, out_hbm, num_hbm, acc,
            semg0, semg1, semi0, semi1):
        c = lax.axis_index("c")
        s = lax.axis_index("s")
        tid = c * _NS + s
        base = tid * ept
        cbase = tid * nchunk

        def phase_a(asv_v, anv_v, ib, nbuf):
            pltpu.sync_copy(asv_hbm, asv_v)
            pltpu.sync_copy(anv_hbm, anv_v)

            def mx(i, m):
                return jnp.maximum(m, anv_v[pl.ds(i * 16, 16)])

            mvec = lax.fori_loop(0, _N // 16, mx, anv_v[pl.ds(0, 16)])
            mn = lax.reduce_max(mvec, (0,))

            def batch(b, _):
                pltpu.sync_copy(edge_hbm.at[:, pl.ds(cbase + b * _B, _B), :],
                                ib)
                eb = base + b * (_B * _C)
                for k in range(_B):
                    for j in range(_C // 16):
                        jj = j * 16
                        rv = ib[0, k, pl.ds(jj, 16)]
                        cv = ib[1, k, pl.ds(jj, 16)]
                        ar = plsc.load_gather(asv_v, [rv])
                        ac = plsc.load_gather(anv_v, [cv])
                        z = ar + ac
                        e = jnp.where(z > 0, z, 0.2 * z)
                        num = jnp.exp(e - jnp.maximum(ar + mn, 0.0))
                        gid = eb + k * _C + jj + lax.iota(jnp.int32, 16)
                        num = jnp.where(gid < e_real, num, 0.0)
                        nbuf[pl.ds(k * _C + jj, 16)] = num
                pltpu.sync_copy(nbuf, num_hbm.at[pl.ds(eb, _B * _C)])
                return 0

            lax.fori_loop(0, nbatch, batch, 0)

        pl.run_scoped(phase_a,
                      pltpu.VMEM((_N,), jnp.float32),
                      pltpu.VMEM((_N,), jnp.float32),
                      pltpu.VMEM((2, _B, _C), jnp.int32),
                      pltpu.VMEM((_B * _C,), jnp.float32))

        def phase_b(fbuf0, fbuf1, ebuf0, ebuf1, nbuf):
            zeros16 = jnp.zeros((16,), jnp.float32)

            def zrow(r, _):
                for k in range(_FX // 16):
                    fbuf0[r, pl.ds(k * 16, 16)] = zeros16
                return 0

            lax.fori_loop(0, _C, zrow, 0)
            nbase = s * _NPT
            for k in range(_NPT // _C):
                pltpu.sync_copy(fbuf0.at[pl.ds(0, _C), :],
                                acc.at[pl.ds(nbase + k * _C, _C), :])
            rem = _NPT % _C
            if rem:
                pltpu.sync_copy(fbuf0.at[pl.ds(0, rem), :],
                                acc.at[pl.ds(nbase + _NPT - rem, rem), :])
            plsc.subcore_barrier()

            pltpu.sync_copy(edge_hbm.at[:, pl.ds(cbase, 1), :], ebuf0)
            pltpu.async_copy(featx_hbm.at[ebuf0.at[1, 0]], fbuf0, semg0)
            pltpu.async_copy(edge_hbm.at[:, pl.ds(cbase + 1, 1), :], ebuf1,
                             semi1)

            def halfstep(g, ebuf_p, fbuf_p, semg_p, semi_p, ebuf_q, fbuf_q,
                         semg_q, semi_q):
                cb = base + g * _C
                pltpu.make_async_copy(featx_hbm.at[ebuf_p.at[1, 0]], fbuf_p,
                                      semg_p).wait()

                @pl.when(lax.rem(g, _B) == 0)
                def _load_nums():
                    pltpu.sync_copy(num_hbm.at[pl.ds(cb, _B * _C)], nbuf)

                @pl.when(g + 1 < nchunk)
                def _prefetch_gather():
                    pltpu.make_async_copy(
                        edge_hbm.at[:, pl.ds(cbase + g + 1, 1), :], ebuf_q,
                        semi_q).wait()
                    pltpu.async_copy(featx_hbm.at[ebuf_q.at[1, 0]], fbuf_q,
                                     semg_q)

                noff = lax.rem(g, _B) * _C

                def group(j, _):
                    jj = j * 16
                    nv = nbuf[pl.ds(noff + jj, 16)]
                    for el in range(16):
                        wv = _lane(nv, el)
                        row = jj + el
                        for k in range(_FX // 16):
                            fbuf_p[row, pl.ds(k * 16, 16)] = (
                                fbuf_p[row, pl.ds(k * 16, 16)] * wv)
                    return 0

                lax.fori_loop(0, _C // 16, group, 0)
                pltpu.sync_copy(fbuf_p, acc.at[ebuf_p.at[0, 0]], add=True)

                @pl.when(g + 2 < nchunk)
                def _prefetch_idx():
                    pltpu.async_copy(edge_hbm.at[:, pl.ds(cbase + g + 2, 1),
                                                 :], ebuf_p, semi_p)

            def body(g2, _):
                g = g2 * 2
                halfstep(g, ebuf0, fbuf0, semg0, semi0, ebuf1, fbuf1, semg1,
                         semi1)
                halfstep(g + 1, ebuf1, fbuf1, semg1, semi1, ebuf0, fbuf0,
                         semg0, semi0)
                return 0

            lax.fori_loop(0, nchunk // 2, body, 0)
            plsc.subcore_barrier()
            pltpu.sync_copy(acc.at[pl.ds(nbase, _NPT), :],
                            out_hbm.at[c, pl.ds(nbase, _NPT), :])

        pl.run_scoped(phase_b,
                      pltpu.VMEM((_C, _FX), jnp.float32),
                      pltpu.VMEM((_C, _FX), jnp.float32),
                      pltpu.VMEM((2, 1, _C), jnp.int32),
                      pltpu.VMEM((2, 1, _C), jnp.int32),
                      pltpu.VMEM((_B * _C,), jnp.float32))

    return agg


def kernel(X, edge_index, W, a_self, a_neigh, bias):
    e_real = edge_index.shape[1]
    quantum = _NW * _C * _B * 2
    e_pad = ((e_real + quantum - 1) // quantum) * quantum
    edges = edge_index
    if e_pad != e_real:
        edges = jnp.pad(edge_index, ((0, 0), (0, e_pad - e_real)))
    edges3 = edges.reshape(2, e_pad // _C, _C)

    featx, asv, anv = pl.pallas_call(
        _prep_body,
        out_shape=[
            jax.ShapeDtypeStruct((_N, _FX), jnp.float32),
            jax.ShapeDtypeStruct((_N, 1), jnp.float32),
            jax.ShapeDtypeStruct((_N, 1), jnp.float32),
        ],
    )(X, W, a_self, a_neigh)

    agg = _make_agg(e_real, e_pad)
    partials, _ = agg(edges3, asv.reshape(-1), anv.reshape(-1), featx)

    out = pl.pallas_call(
        _fin_body,
        out_shape=jax.ShapeDtypeStruct((_N, _F), jnp.float32),
    )(partials, bias.reshape(1, _F))
    return out


# speedup vs baseline: 1.2350x; 1.2350x over previous
import functools

import jax
import jax.numpy as jnp
from jax import lax
from jax.experimental import pallas as pl
from jax.experimental.pallas import tpu as pltpu
from jax.experimental.pallas import tpu_sc as plsc

_N = 10000
_F = 128
_FX = 144
_NC = 2
_NS = 16
_NW = _NC * _NS
_C = 64
_NPT = _N // _NS

_DNUMS = lax.GatherDimensionNumbers(
    offset_dims=(), collapsed_slice_dims=(0,), start_index_map=(0,))


def _lane(vec, i):
    idx = jnp.full((16, 1), i, jnp.int32)
    return lax.gather(vec, idx, _DNUMS, (1,),
                      mode=lax.GatherScatterMode.PROMISE_IN_BOUNDS)


def _prep_body(x_ref, w_ref, asw_ref, anw_ref, featx_ref, asv_ref, anv_ref):
    feat = jnp.dot(x_ref[...], w_ref[...], preferred_element_type=jnp.float32)
    asv = jnp.dot(feat, asw_ref[...], preferred_element_type=jnp.float32)
    anv = jnp.dot(feat, anw_ref[...], preferred_element_type=jnp.float32)
    n = feat.shape[0]
    ext = jnp.concatenate(
        [feat, jnp.ones((n, 1), jnp.float32),
         jnp.zeros((n, _FX - _F - 1), jnp.float32)], axis=1)
    featx_ref[...] = ext
    asv_ref[...] = asv
    anv_ref[...] = anv


def _fin_body(p_ref, b_ref, o_ref):
    t = p_ref[0] + p_ref[1]
    numer = t[:, :_F]
    den = t[:, _F:_F + 1]
    o_ref[...] = jnp.maximum(numer / (den + 1e-9) + b_ref[...], 0.0)


def _make_agg(e_real, e_pad):
    ept = e_pad // _NW
    nchunk = ept // _C
    mesh = plsc.VectorSubcoreMesh(core_axis_name="c", subcore_axis_name="s",
                                  num_cores=_NC)

    @functools.partial(
        pl.kernel,
        out_type=jax.ShapeDtypeStruct((_NC, _N, _FX), jnp.float32),
        mesh=mesh,
        compiler_params=pltpu.CompilerParams(use_tc_tiling_on_sc=False,
                                             needs_layout_passes=False),
        scratch_types=[
            pltpu.VMEM((2, _C), jnp.int32),
            pltpu.VMEM((2, _C), jnp.int32),
            pltpu.VMEM((_C, _FX), jnp.float32),
            pltpu.VMEM((_C, _FX), jnp.float32),
            pltpu.VMEM((_N,), jnp.float32),
            pltpu.VMEM((_N,), jnp.float32),
            pltpu.VMEM_SHARED((_N, _FX), jnp.float32),
            pltpu.SemaphoreType.DMA,
            pltpu.SemaphoreType.DMA,
            pltpu.SemaphoreType.DMA,
            pltpu.SemaphoreType.DMA,
        ])
    def agg(edge_hbm, asv_hbm, anv_hbm, featx_hbm, out_hbm,
            ebuf0, ebuf1, fbuf0, fbuf1, asv_v, anv_v, acc,
            semg0, semg1, semi0, semi1):
        c = lax.axis_index("c")
        s = lax.axis_index("s")
        tid = c * _NS + s
        base = tid * ept

        pltpu.sync_copy(asv_hbm, asv_v)
        pltpu.sync_copy(anv_hbm, anv_v)

        def mx(i, m):
            return jnp.maximum(m, anv_v[pl.ds(i * 16, 16)])

        mvec = lax.fori_loop(0, _N // 16, mx, anv_v[pl.ds(0, 16)])
        mn = lax.reduce_max(mvec, (0,))

        zeros16 = jnp.zeros((16,), jnp.float32)

        def zrow(r, _):
            for k in range(_FX // 16):
                fbuf0[r, pl.ds(k * 16, 16)] = zeros16
            return 0

        lax.fori_loop(0, _C, zrow, 0)
        nbase = s * _NPT
        for k in range(_NPT // _C):
            pltpu.sync_copy(fbuf0.at[pl.ds(0, _C), :],
                            acc.at[pl.ds(nbase + k * _C, _C), :])
        rem = _NPT % _C
        if rem:
            pltpu.sync_copy(fbuf0.at[pl.ds(0, rem), :],
                            acc.at[pl.ds(nbase + _NPT - rem, rem), :])
        plsc.subcore_barrier()

        h = _C // 2
        pltpu.sync_copy(edge_hbm.at[:, pl.ds(base, _C)], ebuf0)
        pltpu.async_copy(featx_hbm.at[ebuf0.at[1, pl.ds(0, h)]],
                         fbuf0.at[pl.ds(0, h), :], semg0)
        pltpu.async_copy(featx_hbm.at[ebuf0.at[1, pl.ds(h, h)]],
                         fbuf0.at[pl.ds(h, h), :], semg0)
        pltpu.async_copy(edge_hbm.at[:, pl.ds(base + _C, _C)], ebuf1, semi1)

        def halfstep(g, ebuf_p, fbuf_p, semg_p, semi_p, ebuf_q, fbuf_q,
                     semg_q, semi_q):
            cb = base + g * _C
            pltpu.make_async_copy(featx_hbm.at[ebuf_p.at[1, pl.ds(0, h)]],
                                  fbuf_p.at[pl.ds(0, h), :], semg_p).wait()
            pltpu.make_async_copy(featx_hbm.at[ebuf_p.at[1, pl.ds(h, h)]],
                                  fbuf_p.at[pl.ds(h, h), :], semg_p).wait()

            @pl.when(g + 1 < nchunk)
            def _prefetch_gather():
                pltpu.make_async_copy(
                    edge_hbm.at[:, pl.ds(cb + _C, _C)], ebuf_q, semi_q).wait()
                pltpu.async_copy(featx_hbm.at[ebuf_q.at[1, pl.ds(0, h)]],
                                 fbuf_q.at[pl.ds(0, h), :], semg_q)
                pltpu.async_copy(featx_hbm.at[ebuf_q.at[1, pl.ds(h, h)]],
                                 fbuf_q.at[pl.ds(h, h), :], semg_q)

            def group(j, _):
                jj = j * 16
                rv = ebuf_p[0, pl.ds(jj, 16)]
                cv = ebuf_p[1, pl.ds(jj, 16)]
                ar = plsc.load_gather(asv_v, [rv])
                ac = plsc.load_gather(anv_v, [cv])
                z = ar + ac
                e = jnp.where(z > 0, z, 0.2 * z)
                num = jnp.exp(e - jnp.maximum(ar + mn, 0.0))
                gid = cb + jj + lax.iota(jnp.int32, 16)
                num = jnp.where(gid < e_real, num, 0.0)
                for el in range(16):
                    wv = _lane(num, el)
                    row = jj + el
                    for k in range(_FX // 16):
                        fbuf_p[row, pl.ds(k * 16, 16)] = (
                            fbuf_p[row, pl.ds(k * 16, 16)] * wv)
                return 0

            for j in range(_C // 16):
                group(j, 0)
            pltpu.sync_copy(fbuf_p, acc.at[ebuf_p.at[0]], add=True)

            @pl.when(g + 2 < nchunk)
            def _prefetch_idx():
                pltpu.async_copy(edge_hbm.at[:, pl.ds(cb + 2 * _C, _C)],
                                 ebuf_p, semi_p)

        def body(g2, _):
            g = g2 * 2
            halfstep(g, ebuf0, fbuf0, semg0, semi0, ebuf1, fbuf1, semg1,
                     semi1)
            halfstep(g + 1, ebuf1, fbuf1, semg1, semi1, ebuf0, fbuf0, semg0,
                     semi0)
            return 0

        lax.fori_loop(0, nchunk // 2, body, 0)
        plsc.subcore_barrier()
        pltpu.sync_copy(acc.at[pl.ds(nbase, _NPT), :],
                        out_hbm.at[c, pl.ds(nbase, _NPT), :])

    return agg


def kernel(X, edge_index, W, a_self, a_neigh, bias):
    e_real = edge_index.shape[1]
    quantum = _NW * _C * 2
    e_pad = ((e_real + quantum - 1) // quantum) * quantum
    edges = edge_index
    if e_pad != e_real:
        edges = jnp.pad(edge_index, ((0, 0), (0, e_pad - e_real)))

    featx, asv, anv = pl.pallas_call(
        _prep_body,
        out_shape=[
            jax.ShapeDtypeStruct((_N, _FX), jnp.float32),
            jax.ShapeDtypeStruct((_N, 1), jnp.float32),
            jax.ShapeDtypeStruct((_N, 1), jnp.float32),
        ],
    )(X, W, a_self, a_neigh)

    agg = _make_agg(e_real, e_pad)
    partials = agg(edges, asv.reshape(-1), anv.reshape(-1), featx)

    out = pl.pallas_call(
        _fin_body,
        out_shape=jax.ShapeDtypeStruct((_N, _F), jnp.float32),
    )(partials, bias.reshape(1, _F))
    return out
